# baseline (speedup 1.0000x reference)
import functools

import jax
import jax.numpy as jnp
from jax import lax
from jax.experimental import pallas as pl
from jax.experimental.pallas import tpu as pltpu
from jax.experimental.pallas import tpu_sc as plsc

N = 10000
NPAD = 10240
E = 320000
NC = 2
NS = 16
NW = NC * NS
CHUNK = 128
NCHUNK = 80
PER_TILE = NCHUNK * CHUNK
EPAD = PER_TILE * NW
ROWS_PER_TILE = NPAD // NS
EPS = 1e-5

_mesh = functools.partial(
    plsc.VectorSubcoreMesh, core_axis_name="c", subcore_axis_name="s")


def _zero_fill(buf, rows, cols):
  zero = jnp.zeros((16,), jnp.float32)
  cpr = cols // 16

  def body(i, _):
    buf[i // cpr, pl.ds((i % cpr) * 16, 16)] = zero
    return 0

  lax.fori_loop(0, rows * cpr, body, 0)


def _make_deg_kernel():
  C = 16

  @functools.partial(
      pl.kernel,
      mesh=_mesh(),
      out_type=jax.ShapeDtypeStruct((NC, NPAD, C), jnp.float32),
      compiler_params=pltpu.CompilerParams(use_tc_tiling_on_sc=False),
      scratch_types=[
          pltpu.VMEM((NCHUNK, CHUNK), jnp.int32),
          pltpu.VMEM((CHUNK, C), jnp.float32),
          pltpu.VMEM((CHUNK, C), jnp.float32),
          pltpu.VMEM_SHARED((NPAD, C), jnp.float32),
          pltpu.SemaphoreType.DMA,
      ],
  )
  def deg_kernel(dst_hbm, out_hbm, dst_v, ones_v, zbuf, acc_sh, sem):
    cid = lax.axis_index("c")
    sid = lax.axis_index("s")
    wid = cid * NS + sid

    _zero_fill(zbuf, CHUNK, C)
    one = jnp.full((16,), 1.0, jnp.float32)

    def fill_ones(i, _):
      ones_v[i, pl.ds(0, 16)] = one
      return 0

    lax.fori_loop(0, CHUNK, fill_ones, 0)

    def zseg(j, _):
      pltpu.sync_copy(zbuf, acc_sh.at[pl.ds(sid * ROWS_PER_TILE + j * CHUNK,
                                            CHUNK)])
      return 0

    lax.fori_loop(0, ROWS_PER_TILE // CHUNK, zseg, 0)
    pltpu.sync_copy(dst_hbm.at[wid], dst_v)
    plsc.subcore_barrier()

    LAG = 4

    def issue_s(j):
      pltpu.async_copy(ones_v, acc_sh.at[dst_v.at[j]], sem, add=True)

    for b in range(LAG):
      issue_s(b)

    def scat(j, _):
      @pl.when(j + LAG < NCHUNK)
      def _():
        issue_s(j + LAG)

      pltpu.make_async_copy(ones_v, acc_sh.at[dst_v.at[j]], sem).wait()
      return 0

    lax.fori_loop(0, NCHUNK, scat, 0)
    plsc.subcore_barrier()

    pltpu.sync_copy(
        acc_sh.at[pl.ds(sid * ROWS_PER_TILE, ROWS_PER_TILE)],
        out_hbm.at[cid, pl.ds(sid * ROWS_PER_TILE, ROWS_PER_TILE)])

  return deg_kernel


def _make_conv_kernel(C):
  @functools.partial(
      pl.kernel,
      mesh=_mesh(),
      out_type=jax.ShapeDtypeStruct((NC, NPAD, C), jnp.float32),
      compiler_params=pltpu.CompilerParams(use_tc_tiling_on_sc=False),
      scratch_types=[
          pltpu.VMEM((NCHUNK, CHUNK), jnp.int32),
          pltpu.VMEM((NCHUNK, CHUNK), jnp.int32),
          pltpu.VMEM((4, CHUNK, C), jnp.float32),
          pltpu.VMEM((CHUNK, C), jnp.float32),
          pltpu.VMEM_SHARED((NPAD, C), jnp.float32),
          [pltpu.SemaphoreType.DMA] * 4,
          [pltpu.SemaphoreType.DMA] * 4,
      ],
  )
  def conv_kernel(y_hbm, src_hbm, dst_hbm, out_hbm,
                  src_v, dst_v, rows_v, zbuf, acc_sh, gsems, ssems):
    cid = lax.axis_index("c")
    sid = lax.axis_index("s")
    wid = cid * NS + sid

    _zero_fill(zbuf, CHUNK, C)

    def zseg(j, _):
      pltpu.sync_copy(zbuf, acc_sh.at[pl.ds(sid * ROWS_PER_TILE + j * CHUNK,
                                            CHUNK)])
      return 0

    lax.fori_loop(0, ROWS_PER_TILE // CHUNK, zseg, 0)
    pltpu.sync_copy(src_hbm.at[wid], src_v)
    pltpu.sync_copy(dst_hbm.at[wid], dst_v)
    plsc.subcore_barrier()

    def issue_g(j, b):
      pltpu.async_copy(y_hbm.at[src_v.at[j]], rows_v.at[b], gsems[b])

    def wait_g(j, b):
      pltpu.make_async_copy(y_hbm.at[src_v.at[j]], rows_v.at[b],
                            gsems[b]).wait()

    def issue_s(j, b):
      pltpu.async_copy(rows_v.at[b], acc_sh.at[dst_v.at[j]], ssems[b],
                       add=True)

    def wait_s(j, b):
      pltpu.make_async_copy(rows_v.at[b], acc_sh.at[dst_v.at[j]],
                            ssems[b]).wait()

    issue_g(0, 0)
    issue_g(1, 1)

    def group(g, _):
      for b in range(4):
        t = g * 4 + b
        bw = (b + 2) % 4

        @pl.when(t >= 2)
        def _():
          wait_s(t - 2, bw)

        @pl.when(t + 2 < NCHUNK)
        def _():
          issue_g(t + 2, bw)

        wait_g(t, b)
        issue_s(t, b)
      return 0

    lax.fori_loop(0, NCHUNK // 4, group, 0)
    wait_s(NCHUNK - 2, (NCHUNK - 2) % 4)
    wait_s(NCHUNK - 1, (NCHUNK - 1) % 4)
    plsc.subcore_barrier()

    pltpu.sync_copy(
        acc_sh.at[pl.ds(sid * ROWS_PER_TILE, ROWS_PER_TILE)],
        out_hbm.at[cid, pl.ds(sid * ROWS_PER_TILE, ROWS_PER_TILE)])

  return conv_kernel


_deg_kernel = _make_deg_kernel()
_conv16 = _make_conv_kernel(16)
_conv32 = _make_conv_kernel(32)


def _tc1_body(degp_ref, x_ref, w1_ref, dinv_ref, y1_ref):
  deg = degp_ref[0] + degp_ref[1] + 1.0
  dinv = lax.rsqrt(deg)
  xw = jnp.dot(x_ref[...], w1_ref[...], preferred_element_type=jnp.float32)
  dinv_ref[...] = dinv
  y1_ref[...] = xw * dinv


def _tc2_body(accp_ref, y1_ref, dinv_ref, w2_ref, cvec_ref, y2_ref):
  acc = accp_ref[0] + accp_ref[1] + y1_ref[...]
  conv = acc * dinv_ref[...] + cvec_ref[0:1, :]
  h = jnp.maximum(conv * cvec_ref[1:2, :] + cvec_ref[2:3, :], 0.0)
  h2 = jnp.dot(h, w2_ref[...], preferred_element_type=jnp.float32)
  y2_ref[...] = h2 * dinv_ref[:, 0:1]


def _tc3_body(accp_ref, y2_ref, dinv_ref, batch_ref, cvec_ref,
              linw_ref, linb_ref, out_ref):
  acc = accp_ref[0] + accp_ref[1] + y2_ref[...]
  conv = acc * dinv_ref[:, 0:1] + cvec_ref[0:1, :]
  h = jnp.maximum(conv * cvec_ref[1:2, :] + cvec_ref[2:3, :], 0.0)
  ones_col = jnp.ones((NPAD, 1), jnp.float32)
  he = jnp.concatenate([h, ones_col], axis=1)
  gids = lax.broadcasted_iota(jnp.int32, (64, NPAD), 0)
  p = (batch_ref[...] == gids).astype(jnp.float32)
  se = jnp.dot(p, he, preferred_element_type=jnp.float32)
  pooled = se[:, :32] / jnp.maximum(se[:, 32:33], 1.0)
  out_ref[...] = jnp.dot(pooled, linw_ref[...],
                         preferred_element_type=jnp.float32) + linb_ref[...]


def kernel(x, edge_index, batch, W1, b1, bn1_w, bn1_b, W2, b2, bn2_w, bn2_b,
           lin_W, lin_b):
  f32 = jnp.float32
  src = edge_index[0].astype(jnp.int32)
  dst = edge_index[1].astype(jnp.int32)
  pad = EPAD - E
  src_p = jnp.concatenate([src, jnp.zeros((pad,), jnp.int32)])
  src_p = src_p.reshape(NW, NCHUNK, CHUNK)
  dst_p = jnp.concatenate([dst, jnp.full((pad,), N, jnp.int32)])
  dst_p = dst_p.reshape(NW, NCHUNK, CHUNK)
  x_p = jnp.concatenate([x, jnp.zeros((NPAD - N, x.shape[1]), f32)])
  batch_p = jnp.concatenate(
      [batch.astype(jnp.int32), jnp.full((NPAD - N,), 64, jnp.int32)])
  batch_p = batch_p.reshape(1, NPAD)

  bn_scale1 = bn1_w * (1.0 / jnp.sqrt(1.0 + EPS))
  bn_scale2 = bn2_w * (1.0 / jnp.sqrt(1.0 + EPS))
  cvec1 = jnp.stack([b1, bn_scale1, bn1_b])
  cvec2 = jnp.stack([b2, bn_scale2, bn2_b])

  degp = _deg_kernel(dst_p)

  dinv, y1 = pl.pallas_call(
      _tc1_body,
      out_shape=(jax.ShapeDtypeStruct((NPAD, 16), f32),
                 jax.ShapeDtypeStruct((NPAD, 16), f32)),
  )(degp, x_p, W1)

  acc1 = _conv16(y1, src_p, dst_p)

  y2 = pl.pallas_call(
      _tc2_body,
      out_shape=jax.ShapeDtypeStruct((NPAD, 32), f32),
  )(acc1, y1, dinv, W2, cvec1)

  acc2 = _conv32(y2, src_p, dst_p)

  out = pl.pallas_call(
      _tc3_body,
      out_shape=jax.ShapeDtypeStruct((64, 64), f32),
  )(acc2, y2, dinv, batch_p, cvec2, lin_W, lin_b.reshape(1, 64))

  return out


# speedup vs baseline: 1.0215x; 1.0215x over previous
import functools

import jax
import jax.numpy as jnp
from jax import lax
from jax.experimental import pallas as pl
from jax.experimental.pallas import tpu as pltpu
from jax.experimental.pallas import tpu_sc as plsc

N = 10000
NPAD = 10240
E = 320000
NC = 2
NS = 16
NW = NC * NS
CHUNK = 128
NCHUNK = 80
PER_TILE = NCHUNK * CHUNK
EPAD = PER_TILE * NW
ROWS_PER_TILE = NPAD // NS
EPS = 1e-5

_mesh = functools.partial(
    plsc.VectorSubcoreMesh, core_axis_name="c", subcore_axis_name="s")


def _zero_fill(buf, rows, cols):
  zero = jnp.zeros((16,), jnp.float32)
  cpr = cols // 16

  def body(i, _):
    buf[i // cpr, pl.ds((i % cpr) * 16, 16)] = zero
    return 0

  lax.fori_loop(0, rows * cpr, body, 0)


def _make_deg_kernel():
  C = 16

  @functools.partial(
      pl.kernel,
      mesh=_mesh(),
      out_type=jax.ShapeDtypeStruct((NC, NPAD, C), jnp.float32),
      compiler_params=pltpu.CompilerParams(use_tc_tiling_on_sc=False),
      scratch_types=[
          pltpu.VMEM((NCHUNK, CHUNK), jnp.int32),
          pltpu.VMEM((CHUNK, C), jnp.float32),
          pltpu.VMEM((CHUNK, C), jnp.float32),
          pltpu.VMEM_SHARED((NPAD, C), jnp.float32),
          pltpu.SemaphoreType.DMA,
      ],
  )
  def deg_kernel(dst_hbm, out_hbm, dst_v, ones_v, zbuf, acc_sh, sem):
    cid = lax.axis_index("c")
    sid = lax.axis_index("s")
    wid = cid * NS + sid

    _zero_fill(zbuf, CHUNK, C)
    one = jnp.full((16,), 1.0, jnp.float32)

    def fill_ones(i, _):
      ones_v[i, pl.ds(0, 16)] = one
      return 0

    lax.fori_loop(0, CHUNK, fill_ones, 0)

    def zseg(j, _):
      pltpu.sync_copy(zbuf, acc_sh.at[pl.ds(sid * ROWS_PER_TILE + j * CHUNK,
                                            CHUNK)])
      return 0

    lax.fori_loop(0, ROWS_PER_TILE // CHUNK, zseg, 0)
    pltpu.sync_copy(dst_hbm.at[wid], dst_v)
    plsc.subcore_barrier()

    LAG = 4

    def issue_s(j):
      pltpu.async_copy(ones_v, acc_sh.at[dst_v.at[j]], sem, add=True)

    for b in range(LAG):
      issue_s(b)

    def scat(j, _):
      @pl.when(j + LAG < NCHUNK)
      def _():
        issue_s(j + LAG)

      pltpu.make_async_copy(ones_v, acc_sh.at[dst_v.at[j]], sem).wait()
      return 0

    lax.fori_loop(0, NCHUNK, scat, 0)
    plsc.subcore_barrier()

    pltpu.sync_copy(
        acc_sh.at[pl.ds(sid * ROWS_PER_TILE, ROWS_PER_TILE)],
        out_hbm.at[cid, pl.ds(sid * ROWS_PER_TILE, ROWS_PER_TILE)])

  return deg_kernel


def _make_conv_kernel(C):
  @functools.partial(
      pl.kernel,
      mesh=_mesh(),
      out_type=jax.ShapeDtypeStruct((NC, NPAD, C), jnp.float32),
      compiler_params=pltpu.CompilerParams(use_tc_tiling_on_sc=False),
      scratch_types=[
          pltpu.VMEM((NCHUNK, CHUNK), jnp.int32),
          pltpu.VMEM((NCHUNK, CHUNK), jnp.int32),
          pltpu.VMEM((4, CHUNK, C), jnp.float32),
          pltpu.VMEM((CHUNK, C), jnp.float32),
          pltpu.VMEM_SHARED((NPAD, C), jnp.float32),
          [pltpu.SemaphoreType.DMA] * 4,
          [pltpu.SemaphoreType.DMA] * 4,
      ],
  )
  def conv_kernel(y_hbm, src_hbm, dst_hbm, out_hbm,
                  src_v, dst_v, rows_v, zbuf, acc_sh, gsems, ssems):
    cid = lax.axis_index("c")
    sid = lax.axis_index("s")
    wid = cid * NS + sid

    _zero_fill(zbuf, CHUNK, C)

    def zseg(j, _):
      pltpu.sync_copy(zbuf, acc_sh.at[pl.ds(sid * ROWS_PER_TILE + j * CHUNK,
                                            CHUNK)])
      return 0

    lax.fori_loop(0, ROWS_PER_TILE // CHUNK, zseg, 0)
    pltpu.sync_copy(src_hbm.at[wid], src_v)
    pltpu.sync_copy(dst_hbm.at[wid], dst_v)
    plsc.subcore_barrier()

    def issue_g(j, b):
      pltpu.async_copy(y_hbm.at[src_v.at[j]], rows_v.at[b], gsems[b])

    def wait_g(j, b):
      pltpu.make_async_copy(y_hbm.at[src_v.at[j]], rows_v.at[b],
                            gsems[b]).wait()

    def issue_s(j, b):
      pltpu.async_copy(rows_v.at[b], acc_sh.at[dst_v.at[j]], ssems[b],
                       add=True)

    def wait_s(j, b):
      pltpu.make_async_copy(rows_v.at[b], acc_sh.at[dst_v.at[j]],
                            ssems[b]).wait()

    issue_g(0, 0)
    issue_g(1, 1)

    def group(g, _):
      for b in range(4):
        t = g * 4 + b
        bw = (b + 2) % 4

        @pl.when(t >= 2)
        def _():
          wait_s(t - 2, bw)

        @pl.when(t + 2 < NCHUNK)
        def _():
          issue_g(t + 2, bw)

        wait_g(t, b)
        issue_s(t, b)
      return 0

    lax.fori_loop(0, NCHUNK // 4, group, 0)
    wait_s(NCHUNK - 2, (NCHUNK - 2) % 4)
    wait_s(NCHUNK - 1, (NCHUNK - 1) % 4)
    plsc.subcore_barrier()

    pltpu.sync_copy(
        acc_sh.at[pl.ds(sid * ROWS_PER_TILE, ROWS_PER_TILE)],
        out_hbm.at[cid, pl.ds(sid * ROWS_PER_TILE, ROWS_PER_TILE)])

  return conv_kernel


_deg_kernel = _make_deg_kernel()
_conv16 = _make_conv_kernel(16)
_conv32 = _make_conv_kernel(32)


def _tc1_body(degp_ref, x_ref, w1_ref, dinv_ref, y1_ref):
  deg = degp_ref[0] + degp_ref[1] + 1.0
  dinv = lax.rsqrt(deg)
  xw = jnp.dot(x_ref[...], w1_ref[...], preferred_element_type=jnp.float32)
  dinv_ref[...] = dinv
  y1_ref[...] = xw * dinv


def _tc2_body(accp_ref, y1_ref, dinv_ref, w2_ref, cvec_ref, y2_ref):
  acc = accp_ref[0] + accp_ref[1] + y1_ref[...]
  conv = acc * dinv_ref[...] + cvec_ref[0:1, :]
  h = jnp.maximum(conv * cvec_ref[1:2, :] + cvec_ref[2:3, :], 0.0)
  h2 = jnp.dot(h, w2_ref[...], preferred_element_type=jnp.float32)
  y2_ref[...] = h2 * dinv_ref[:, 0:1]


def _tc3_body(accp_ref, y2_ref, dinv_ref, batch_ref, cvec_ref,
              linw_ref, linb_ref, out_ref):
  acc = accp_ref[0] + accp_ref[1] + y2_ref[...]
  conv = acc * dinv_ref[:, 0:1] + cvec_ref[0:1, :]
  h = jnp.maximum(conv * cvec_ref[1:2, :] + cvec_ref[2:3, :], 0.0)
  ones_col = jnp.ones((NPAD, 1), jnp.float32)
  he = jnp.concatenate([h, ones_col], axis=1)
  gids = lax.broadcasted_iota(jnp.int32, (64, NPAD), 0)
  p = (batch_ref[...] == gids).astype(jnp.float32)
  se = jnp.dot(p, he, preferred_element_type=jnp.float32)
  pooled = se[:, :32] / jnp.maximum(se[:, 32:33], 1.0)
  out_ref[...] = jnp.dot(pooled, linw_ref[...],
                         preferred_element_type=jnp.float32) + linb_ref[...]


def kernel(x, edge_index, batch, W1, b1, bn1_w, bn1_b, W2, b2, bn2_w, bn2_b,
           lin_W, lin_b):
  f32 = jnp.float32
  src = edge_index[0].astype(jnp.int32)
  dst = edge_index[1].astype(jnp.int32)
  pad = EPAD - E
  src_p = jnp.concatenate([src, jnp.zeros((pad,), jnp.int32)])
  src_p = src_p.reshape(NW, NCHUNK, CHUNK)
  pad_dst = N + (jnp.arange(pad, dtype=jnp.int32) % (NPAD - N))
  dst_p = jnp.concatenate([dst, pad_dst])
  dst_p = dst_p.reshape(NW, NCHUNK, CHUNK)
  x_p = jnp.concatenate([x, jnp.zeros((NPAD - N, x.shape[1]), f32)])
  batch_p = jnp.concatenate(
      [batch.astype(jnp.int32), jnp.full((NPAD - N,), 64, jnp.int32)])
  batch_p = batch_p.reshape(1, NPAD)

  bn_scale1 = bn1_w * (1.0 / jnp.sqrt(1.0 + EPS))
  bn_scale2 = bn2_w * (1.0 / jnp.sqrt(1.0 + EPS))
  cvec1 = jnp.stack([b1, bn_scale1, bn1_b])
  cvec2 = jnp.stack([b2, bn_scale2, bn2_b])

  degp = _deg_kernel(dst_p)

  dinv, y1 = pl.pallas_call(
      _tc1_body,
      out_shape=(jax.ShapeDtypeStruct((NPAD, 16), f32),
                 jax.ShapeDtypeStruct((NPAD, 16), f32)),
  )(degp, x_p, W1)

  acc1 = _conv16(y1, src_p, dst_p)

  y2 = pl.pallas_call(
      _tc2_body,
      out_shape=jax.ShapeDtypeStruct((NPAD, 32), f32),
  )(acc1, y1, dinv, W2, cvec1)

  acc2 = _conv32(y2, src_p, dst_p)

  out = pl.pallas_call(
      _tc3_body,
      out_shape=jax.ShapeDtypeStruct((64, 64), f32),
  )(acc2, y2, dinv, batch_p, cvec2, lin_W, lin_b.reshape(1, 64))

  return out


# speedup vs baseline: 1.7191x; 1.6830x over previous
import functools

import jax
import jax.numpy as jnp
from jax import lax
from jax.experimental import pallas as pl
from jax.experimental.pallas import tpu as pltpu
from jax.experimental.pallas import tpu_sc as plsc

N = 10000
NPAD = 10240
E = 320000
NC = 2
NS = 16
NW = NC * NS
CHUNK = 128
NCHUNK = 80
PER_TILE = NCHUNK * CHUNK
EPAD = PER_TILE * NW
ROWS_PER_TILE = NPAD // NS
EPS = 1e-5

_mesh = functools.partial(
    plsc.VectorSubcoreMesh, core_axis_name="c", subcore_axis_name="s")


def _zero_fill(buf, rows, cols):
  zero = jnp.zeros((16,), jnp.float32)
  cpr = cols // 16

  def body(i, _):
    buf[i // cpr, pl.ds((i % cpr) * 16, 16)] = zero
    return 0

  lax.fori_loop(0, rows * cpr, body, 0)


def _make_deg_kernel():
  C = 16

  @functools.partial(
      pl.kernel,
      mesh=_mesh(),
      out_type=jax.ShapeDtypeStruct((NC, NPAD, C), jnp.float32),
      compiler_params=pltpu.CompilerParams(use_tc_tiling_on_sc=False),
      scratch_types=[
          pltpu.VMEM((NCHUNK, CHUNK), jnp.int32),
          pltpu.VMEM((CHUNK, C), jnp.float32),
          pltpu.VMEM((CHUNK, C), jnp.float32),
          pltpu.VMEM_SHARED((NPAD, C), jnp.float32),
          pltpu.SemaphoreType.DMA,
      ],
  )
  def deg_kernel(dst_hbm, out_hbm, dst_v, ones_v, zbuf, acc_sh, sem):
    cid = lax.axis_index("c")
    sid = lax.axis_index("s")
    wid = cid * NS + sid

    _zero_fill(zbuf, CHUNK, C)
    one = jnp.full((16,), 1.0, jnp.float32)

    def fill_ones(i, _):
      ones_v[i, pl.ds(0, 16)] = one
      return 0

    lax.fori_loop(0, CHUNK, fill_ones, 0)

    def zseg(j, _):
      pltpu.sync_copy(zbuf, acc_sh.at[pl.ds(sid * ROWS_PER_TILE + j * CHUNK,
                                            CHUNK)])
      return 0

    lax.fori_loop(0, ROWS_PER_TILE // CHUNK, zseg, 0)
    pltpu.sync_copy(dst_hbm.at[wid], dst_v)
    plsc.subcore_barrier()

    LAG = 4

    def issue_s(j):
      pltpu.async_copy(ones_v, acc_sh.at[dst_v.at[j]], sem, add=True)

    for b in range(LAG):
      issue_s(b)

    def scat(j, _):
      @pl.when(j + LAG < NCHUNK)
      def _():
        issue_s(j + LAG)

      pltpu.make_async_copy(ones_v, acc_sh.at[dst_v.at[j]], sem).wait()
      return 0

    lax.fori_loop(0, NCHUNK, scat, 0)
    plsc.subcore_barrier()

    pltpu.sync_copy(
        acc_sh.at[pl.ds(sid * ROWS_PER_TILE, ROWS_PER_TILE)],
        out_hbm.at[cid, pl.ds(sid * ROWS_PER_TILE, ROWS_PER_TILE)])

  return deg_kernel


def _make_conv_kernel(C):
  @functools.partial(
      pl.kernel,
      mesh=_mesh(),
      out_type=jax.ShapeDtypeStruct((NC, NPAD, C), jnp.float32),
      compiler_params=pltpu.CompilerParams(use_tc_tiling_on_sc=False),
      scratch_types=[
          pltpu.VMEM((NCHUNK, CHUNK), jnp.int32),
          pltpu.VMEM((NCHUNK, CHUNK), jnp.int32),
          pltpu.VMEM((4, CHUNK, C), jnp.float32),
          pltpu.VMEM((CHUNK, C), jnp.float32),
          pltpu.VMEM_SHARED((NPAD, C), jnp.float32),
          [pltpu.SemaphoreType.DMA] * 4,
          [pltpu.SemaphoreType.DMA] * 4,
      ],
  )
  def conv_kernel(y_hbm, src_hbm, dst_hbm, out_hbm,
                  src_v, dst_v, rows_v, zbuf, acc_sh, gsems, ssems):
    cid = lax.axis_index("c")
    sid = lax.axis_index("s")
    wid = cid * NS + sid

    _zero_fill(zbuf, CHUNK, C)

    def zseg(j, _):
      pltpu.sync_copy(zbuf, acc_sh.at[pl.ds(sid * ROWS_PER_TILE + j * CHUNK,
                                            CHUNK)])
      return 0

    lax.fori_loop(0, ROWS_PER_TILE // CHUNK, zseg, 0)
    pltpu.sync_copy(src_hbm.at[wid], src_v)
    pltpu.sync_copy(dst_hbm.at[wid], dst_v)
    plsc.subcore_barrier()

    def issue_g(j, b):
      pltpu.async_copy(y_hbm.at[src_v.at[j]], rows_v.at[b], gsems[b])

    def wait_g(j, b):
      pltpu.make_async_copy(y_hbm.at[src_v.at[j]], rows_v.at[b],
                            gsems[b]).wait()

    def issue_s(j, b):
      pltpu.async_copy(rows_v.at[b], acc_sh.at[dst_v.at[j]], ssems[b],
                       add=True)

    def wait_s(j, b):
      pltpu.make_async_copy(rows_v.at[b], acc_sh.at[dst_v.at[j]],
                            ssems[b]).wait()

    issue_g(0, 0)
    issue_g(1, 1)

    def group(g, _):
      for b in range(4):
        t = g * 4 + b
        bw = (b + 2) % 4

        @pl.when(t >= 2)
        def _():
          wait_s(t - 2, bw)

        @pl.when(t + 2 < NCHUNK)
        def _():
          issue_g(t + 2, bw)

        wait_g(t, b)
        issue_s(t, b)
      return 0

    lax.fori_loop(0, NCHUNK // 4, group, 0)
    wait_s(NCHUNK - 2, (NCHUNK - 2) % 4)
    wait_s(NCHUNK - 1, (NCHUNK - 1) % 4)
    plsc.subcore_barrier()

    pltpu.sync_copy(
        acc_sh.at[pl.ds(sid * ROWS_PER_TILE, ROWS_PER_TILE)],
        out_hbm.at[cid, pl.ds(sid * ROWS_PER_TILE, ROWS_PER_TILE)])

  return conv_kernel


_deg_kernel = _make_deg_kernel()
_conv16 = _make_conv_kernel(16)
_conv32 = _make_conv_kernel(32)


def _tc1_body(degp_ref, x_ref, w1_ref, dinv_ref, y1_ref):
  deg = degp_ref[0] + degp_ref[1] + 1.0
  dinv = lax.rsqrt(deg)
  xw = jnp.dot(x_ref[...], w1_ref[...], preferred_element_type=jnp.float32)
  dinv_ref[...] = dinv
  y1_ref[...] = xw * dinv


def _tc2_body(accp_ref, y1_ref, dinv_ref, w2_ref, cvec_ref, y2_ref):
  acc = accp_ref[0] + accp_ref[1] + y1_ref[...]
  conv = acc * dinv_ref[...] + cvec_ref[0:1, :]
  h = jnp.maximum(conv * cvec_ref[1:2, :] + cvec_ref[2:3, :], 0.0)
  h2 = jnp.dot(h, w2_ref[...], preferred_element_type=jnp.float32)
  y2_ref[...] = h2 * dinv_ref[:, 0:1]


def _tc3_body(accp_ref, y2_ref, dinv_ref, batch_ref, cvec_ref,
              linw_ref, linb_ref, out_ref):
  acc = accp_ref[0] + accp_ref[1] + y2_ref[...]
  conv = acc * dinv_ref[:, 0:1] + cvec_ref[0:1, :]
  h = jnp.maximum(conv * cvec_ref[1:2, :] + cvec_ref[2:3, :], 0.0)
  ones_col = jnp.ones((NPAD, 1), jnp.float32)
  he = jnp.concatenate([h, ones_col], axis=1)
  gids = lax.broadcasted_iota(jnp.int32, (64, NPAD), 0)
  p = (batch_ref[...] == gids).astype(jnp.float32)
  se = jnp.dot(p, he, preferred_element_type=jnp.float32)
  pooled = se[:, :32] / jnp.maximum(se[:, 32:33], 1.0)
  out_ref[...] = jnp.dot(pooled, linw_ref[...],
                         preferred_element_type=jnp.float32) + linb_ref[...]


def kernel(x, edge_index, batch, W1, b1, bn1_w, bn1_b, W2, b2, bn2_w, bn2_b,
           lin_W, lin_b):
  f32 = jnp.float32
  src = edge_index[0].astype(jnp.int32)
  dst = edge_index[1].astype(jnp.int32)
  pad = EPAD - E
  pad_rows = N + (jnp.arange(pad, dtype=jnp.int32) % (NPAD - N))
  src_p = jnp.concatenate([src, pad_rows])
  src_p = src_p.reshape(NW, NCHUNK, CHUNK)
  dst_p = jnp.concatenate([dst, pad_rows])
  dst_p = dst_p.reshape(NW, NCHUNK, CHUNK)
  x_p = jnp.concatenate([x, jnp.zeros((NPAD - N, x.shape[1]), f32)])
  batch_p = jnp.concatenate(
      [batch.astype(jnp.int32), jnp.full((NPAD - N,), 64, jnp.int32)])
  batch_p = batch_p.reshape(1, NPAD)

  bn_scale1 = bn1_w * (1.0 / jnp.sqrt(1.0 + EPS))
  bn_scale2 = bn2_w * (1.0 / jnp.sqrt(1.0 + EPS))
  cvec1 = jnp.stack([b1, bn_scale1, bn1_b])
  cvec2 = jnp.stack([b2, bn_scale2, bn2_b])

  degp = _deg_kernel(dst_p)

  dinv, y1 = pl.pallas_call(
      _tc1_body,
      out_shape=(jax.ShapeDtypeStruct((NPAD, 16), f32),
                 jax.ShapeDtypeStruct((NPAD, 16), f32)),
  )(degp, x_p, W1)

  acc1 = _conv16(y1, src_p, dst_p)

  y2 = pl.pallas_call(
      _tc2_body,
      out_shape=jax.ShapeDtypeStruct((NPAD, 32), f32),
  )(acc1, y1, dinv, W2, cvec1)

  acc2 = _conv32(y2, src_p, dst_p)

  out = pl.pallas_call(
      _tc3_body,
      out_shape=jax.ShapeDtypeStruct((64, 64), f32),
  )(acc2, y2, dinv, batch_p, cvec2, lin_W, lin_b.reshape(1, 64))

  return out


# speedup vs baseline: 1.7713x; 1.0303x over previous
import functools

import jax
import jax.numpy as jnp
from jax import lax
from jax.experimental import pallas as pl
from jax.experimental.pallas import tpu as pltpu
from jax.experimental.pallas import tpu_sc as plsc

N = 10000
NPAD = 10240
E = 320000
NC = 2
NS = 16
NW = NC * NS
CHUNK = 128
NCHUNK = 80
PER_TILE = NCHUNK * CHUNK
EPAD = PER_TILE * NW
ROWS_PER_TILE = NPAD // NS
EPS = 1e-5

_mesh = functools.partial(
    plsc.VectorSubcoreMesh, core_axis_name="c", subcore_axis_name="s")


def _zero_fill(buf, rows, cols):
  zero = jnp.zeros((16,), jnp.float32)
  cpr = cols // 16

  def body(i, _):
    buf[i // cpr, pl.ds((i % cpr) * 16, 16)] = zero
    return 0

  lax.fori_loop(0, rows * cpr, body, 0)


def _newton_rsqrt(d):
  half = d * 0.5
  i = plsc.bitcast(d, jnp.int32)
  i = jnp.int32(0x5F3759DF) - lax.shift_right_logical(i, 1)
  y = plsc.bitcast(i, jnp.float32)
  for _ in range(3):
    y = y * (1.5 - half * y * y)
  return y


def _make_fused_conv1_kernel():
  C = 16
  PACK = NPAD // 16
  PROWS = PACK // NS

  @functools.partial(
      pl.kernel,
      mesh=_mesh(),
      out_type=(jax.ShapeDtypeStruct((NC, NPAD, C), jnp.float32),
                jax.ShapeDtypeStruct((NPAD, C), jnp.float32),
                jax.ShapeDtypeStruct((NPAD, C), jnp.float32)),
      compiler_params=pltpu.CompilerParams(use_tc_tiling_on_sc=False,
                                           needs_layout_passes=False),
      scratch_types=[
          pltpu.VMEM((2, NCHUNK, CHUNK), jnp.int32),
          pltpu.VMEM((NCHUNK, CHUNK), jnp.int32),
          pltpu.VMEM((PACK, 16), jnp.float32),
          pltpu.VMEM((ROWS_PER_TILE, C), jnp.float32),
          pltpu.VMEM((ROWS_PER_TILE, C), jnp.float32),
          pltpu.VMEM((ROWS_PER_TILE, C), jnp.float32),
          pltpu.VMEM((4, CHUNK, C), jnp.float32),
          pltpu.VMEM((CHUNK, C), jnp.float32),
          pltpu.VMEM((PACK // CHUNK, CHUNK), jnp.int32),
          pltpu.VMEM_SHARED((PACK, 16), jnp.float32),
          pltpu.VMEM_SHARED((NPAD, C), jnp.float32),
          pltpu.VMEM_SHARED((NPAD, C), jnp.float32),
          [pltpu.SemaphoreType.DMA] * 4,
          [pltpu.SemaphoreType.DMA] * 4,
      ],
  )
  def fused_kernel(xw_hbm, src_hbm, dst_hbm,
                   acc_hbm, dinv_hbm, y1_hbm,
                   dst_v, src_v, deg_v, xw_v, y1_v, dinv_v, rows_v, zbuf,
                   iid_v, degp_sh, y1_sh, acc_sh, gsems, ssems):
    cid = lax.axis_index("c")
    sid = lax.axis_index("s")
    wid = cid * NS + sid
    nbase = sid * ROWS_PER_TILE

    _zero_fill(zbuf, CHUNK, C)
    _zero_fill(deg_v, PACK, 16)

    lanes = lax.iota(jnp.int32, 16)

    def fill_iid(i, _):
      iid_v[i // 8, pl.ds((i % 8) * 16, 16)] = lanes + i * 16
      return 0

    lax.fori_loop(0, PACK // 16, fill_iid, 0)

    pltpu.sync_copy(zbuf.at[pl.ds(0, PROWS)], degp_sh.at[pl.ds(sid * PROWS,
                                                               PROWS)])

    def zseg(j, _):
      pltpu.sync_copy(zbuf, acc_sh.at[pl.ds(nbase + j * CHUNK, CHUNK)])
      return 0

    lax.fori_loop(0, ROWS_PER_TILE // CHUNK, zseg, 0)

    pltpu.sync_copy(dst_hbm.at[sid], dst_v.at[0])
    pltpu.sync_copy(dst_hbm.at[NS + sid], dst_v.at[1])
    pltpu.sync_copy(src_hbm.at[wid], src_v)
    pltpu.sync_copy(xw_hbm.at[pl.ds(nbase, ROWS_PER_TILE)], xw_v)

    ones16 = jnp.full((16,), 1.0, jnp.float32)
    for h in range(2):

      def dcount(i, _, h=h):
        ids = dst_v[h, i // 8, pl.ds((i % 8) * 16, 16)]
        plsc.addupdate_scatter(
            deg_v, [lax.shift_right_logical(ids, 4), ids & 15], ones16)
        return 0

      lax.fori_loop(0, NCHUNK * 8, dcount, 0)

    plsc.subcore_barrier()

    def comb(r, _):
      pltpu.sync_copy(deg_v.at[pl.ds(r * CHUNK, CHUNK)],
                      degp_sh.at[iid_v.at[r]], add=True)
      return 0

    lax.fori_loop(0, PACK // CHUNK, comb, 0)
    plsc.subcore_barrier()

    pltpu.sync_copy(degp_sh.at[pl.ds(sid * PROWS, PROWS)],
                    deg_v.at[pl.ds(0, PROWS)])

    def newt(r, _):
      deg_v[r, pl.ds(0, 16)] = _newton_rsqrt(deg_v[r, pl.ds(0, 16)] + 1.0)
      return 0

    lax.fori_loop(0, PROWS, newt, 0)

    def expand(i, _):
      ridx = jnp.full((16,), 0, jnp.int32) + lax.shift_right_logical(i, 4)
      cidx = jnp.full((16,), 0, jnp.int32) + (i & 15)
      dvec = plsc.load_gather(deg_v, [ridx, cidx])
      dinv_v[i, pl.ds(0, 16)] = dvec
      y1_v[i, pl.ds(0, 16)] = xw_v[i, pl.ds(0, 16)] * dvec
      return 0

    lax.fori_loop(0, ROWS_PER_TILE, expand, 0)

    pltpu.sync_copy(y1_v, y1_sh.at[pl.ds(nbase, ROWS_PER_TILE)])

    @pl.when(cid == 0)
    def _():
      pltpu.sync_copy(y1_v, y1_hbm.at[pl.ds(nbase, ROWS_PER_TILE)])
      pltpu.sync_copy(dinv_v, dinv_hbm.at[pl.ds(nbase, ROWS_PER_TILE)])

    plsc.subcore_barrier()

    def issue_g(j, b):
      pltpu.async_copy(y1_sh.at[src_v.at[j]], rows_v.at[b], gsems[b])

    def wait_g(j, b):
      pltpu.make_async_copy(y1_sh.at[src_v.at[j]], rows_v.at[b],
                            gsems[b]).wait()

    def issue_s(j, b):
      pltpu.async_copy(rows_v.at[b], acc_sh.at[dst_v.at[cid, j]], ssems[b],
                       add=True)

    def wait_s(j, b):
      pltpu.make_async_copy(rows_v.at[b], acc_sh.at[dst_v.at[cid, j]],
                            ssems[b]).wait()

    issue_g(0, 0)
    issue_g(1, 1)

    def group(g, _):
      for b in range(4):
        t = g * 4 + b
        bw = (b + 2) % 4

        @pl.when(t >= 2)
        def _():
          wait_s(t - 2, bw)

        @pl.when(t + 2 < NCHUNK)
        def _():
          issue_g(t + 2, bw)

        wait_g(t, b)
        issue_s(t, b)
      return 0

    lax.fori_loop(0, NCHUNK // 4, group, 0)
    wait_s(NCHUNK - 2, (NCHUNK - 2) % 4)
    wait_s(NCHUNK - 1, (NCHUNK - 1) % 4)
    plsc.subcore_barrier()

    pltpu.sync_copy(
        acc_sh.at[pl.ds(nbase, ROWS_PER_TILE)],
        acc_hbm.at[cid, pl.ds(nbase, ROWS_PER_TILE)])

  return fused_kernel


def _make_conv_kernel(C):
  @functools.partial(
      pl.kernel,
      mesh=_mesh(),
      out_type=jax.ShapeDtypeStruct((NC, NPAD, C), jnp.float32),
      compiler_params=pltpu.CompilerParams(use_tc_tiling_on_sc=False),
      scratch_types=[
          pltpu.VMEM((NCHUNK, CHUNK), jnp.int32),
          pltpu.VMEM((NCHUNK, CHUNK), jnp.int32),
          pltpu.VMEM((4, CHUNK, C), jnp.float32),
          pltpu.VMEM((CHUNK, C), jnp.float32),
          pltpu.VMEM_SHARED((NPAD, C), jnp.float32),
          [pltpu.SemaphoreType.DMA] * 4,
          [pltpu.SemaphoreType.DMA] * 4,
      ],
  )
  def conv_kernel(y_hbm, src_hbm, dst_hbm, out_hbm,
                  src_v, dst_v, rows_v, zbuf, acc_sh, gsems, ssems):
    cid = lax.axis_index("c")
    sid = lax.axis_index("s")
    wid = cid * NS + sid

    _zero_fill(zbuf, CHUNK, C)

    def zseg(j, _):
      pltpu.sync_copy(zbuf, acc_sh.at[pl.ds(sid * ROWS_PER_TILE + j * CHUNK,
                                            CHUNK)])
      return 0

    lax.fori_loop(0, ROWS_PER_TILE // CHUNK, zseg, 0)
    pltpu.sync_copy(src_hbm.at[wid], src_v)
    pltpu.sync_copy(dst_hbm.at[wid], dst_v)
    plsc.subcore_barrier()

    def issue_g(j, b):
      pltpu.async_copy(y_hbm.at[src_v.at[j]], rows_v.at[b], gsems[b])

    def wait_g(j, b):
      pltpu.make_async_copy(y_hbm.at[src_v.at[j]], rows_v.at[b],
                            gsems[b]).wait()

    def issue_s(j, b):
      pltpu.async_copy(rows_v.at[b], acc_sh.at[dst_v.at[j]], ssems[b],
                       add=True)

    def wait_s(j, b):
      pltpu.make_async_copy(rows_v.at[b], acc_sh.at[dst_v.at[j]],
                            ssems[b]).wait()

    issue_g(0, 0)
    issue_g(1, 1)

    def group(g, _):
      for b in range(4):
        t = g * 4 + b
        bw = (b + 2) % 4

        @pl.when(t >= 2)
        def _():
          wait_s(t - 2, bw)

        @pl.when(t + 2 < NCHUNK)
        def _():
          issue_g(t + 2, bw)

        wait_g(t, b)
        issue_s(t, b)
      return 0

    lax.fori_loop(0, NCHUNK // 4, group, 0)
    wait_s(NCHUNK - 2, (NCHUNK - 2) % 4)
    wait_s(NCHUNK - 1, (NCHUNK - 1) % 4)
    plsc.subcore_barrier()

    pltpu.sync_copy(
        acc_sh.at[pl.ds(sid * ROWS_PER_TILE, ROWS_PER_TILE)],
        out_hbm.at[cid, pl.ds(sid * ROWS_PER_TILE, ROWS_PER_TILE)])

  return conv_kernel


_fused1 = _make_fused_conv1_kernel()
_conv32 = _make_conv_kernel(32)


def _tc0_body(x_ref, w1_ref, xw_ref):
  xw_ref[...] = jnp.dot(x_ref[...], w1_ref[...],
                        preferred_element_type=jnp.float32)


def _tc2_body(accp_ref, y1_ref, dinv_ref, w2_ref, cvec_ref, y2_ref):
  acc = accp_ref[0] + accp_ref[1] + y1_ref[...]
  conv = acc * dinv_ref[...] + cvec_ref[0:1, :]
  h = jnp.maximum(conv * cvec_ref[1:2, :] + cvec_ref[2:3, :], 0.0)
  h2 = jnp.dot(h, w2_ref[...], preferred_element_type=jnp.float32)
  y2_ref[...] = h2 * dinv_ref[:, 0:1]


def _tc3_body(accp_ref, y2_ref, dinv_ref, batch_ref, cvec_ref,
              linw_ref, linb_ref, out_ref):
  acc = accp_ref[0] + accp_ref[1] + y2_ref[...]
  conv = acc * dinv_ref[:, 0:1] + cvec_ref[0:1, :]
  h = jnp.maximum(conv * cvec_ref[1:2, :] + cvec_ref[2:3, :], 0.0)
  ones_col = jnp.ones((NPAD, 1), jnp.float32)
  he = jnp.concatenate([h, ones_col], axis=1)
  gids = lax.broadcasted_iota(jnp.int32, (64, NPAD), 0)
  p = (batch_ref[...] == gids).astype(jnp.float32)
  se = jnp.dot(p, he, preferred_element_type=jnp.float32)
  pooled = se[:, :32] / jnp.maximum(se[:, 32:33], 1.0)
  out_ref[...] = jnp.dot(pooled, linw_ref[...],
                         preferred_element_type=jnp.float32) + linb_ref[...]


def kernel(x, edge_index, batch, W1, b1, bn1_w, bn1_b, W2, b2, bn2_w, bn2_b,
           lin_W, lin_b):
  f32 = jnp.float32
  src = edge_index[0].astype(jnp.int32)
  dst = edge_index[1].astype(jnp.int32)
  pad = EPAD - E
  pad_rows = N + (jnp.arange(pad, dtype=jnp.int32) % (NPAD - N))
  src_p = jnp.concatenate([src, pad_rows])
  src_p = src_p.reshape(NW, NCHUNK, CHUNK)
  dst_p = jnp.concatenate([dst, pad_rows])
  dst_p = dst_p.reshape(NW, NCHUNK, CHUNK)
  x_p = jnp.concatenate([x, jnp.zeros((NPAD - N, x.shape[1]), f32)])
  batch_p = jnp.concatenate(
      [batch.astype(jnp.int32), jnp.full((NPAD - N,), 64, jnp.int32)])
  batch_p = batch_p.reshape(1, NPAD)

  bn_scale1 = bn1_w * (1.0 / jnp.sqrt(1.0 + EPS))
  bn_scale2 = bn2_w * (1.0 / jnp.sqrt(1.0 + EPS))
  cvec1 = jnp.stack([b1, bn_scale1, bn1_b])
  cvec2 = jnp.stack([b2, bn_scale2, bn2_b])

  xw1 = pl.pallas_call(
      _tc0_body,
      out_shape=jax.ShapeDtypeStruct((NPAD, 16), f32),
  )(x_p, W1)

  acc1, dinv, y1 = _fused1(xw1, src_p, dst_p)

  y2 = pl.pallas_call(
      _tc2_body,
      out_shape=jax.ShapeDtypeStruct((NPAD, 32), f32),
  )(acc1, y1, dinv, W2, cvec1)

  acc2 = _conv32(y2, src_p, dst_p)

  out = pl.pallas_call(
      _tc3_body,
      out_shape=jax.ShapeDtypeStruct((64, 64), f32),
  )(acc2, y2, dinv, batch_p, cvec2, lin_W, lin_b.reshape(1, 64))

  return out


# speedup vs baseline: 1.9257x; 1.0872x over previous
import functools

import jax
import jax.numpy as jnp
from jax import lax
from jax.experimental import pallas as pl
from jax.experimental.pallas import tpu as pltpu
from jax.experimental.pallas import tpu_sc as plsc

N = 10000
NPAD = 10240
E = 320000
NC = 2
NS = 16
NW = NC * NS
CHUNK = 128
NCHUNK = 80
PER_TILE = NCHUNK * CHUNK
EPT = E // NW
SYNTH = PER_TILE - EPT
ROWS_PER_TILE = NPAD // NS
EPS = 1e-5

_mesh = functools.partial(
    plsc.VectorSubcoreMesh, core_axis_name="c", subcore_axis_name="s")


def _zero_fill(buf, rows, cols):
  zero = jnp.zeros((16,), jnp.float32)
  cpr = cols // 16

  def body(i, _):
    buf[i // cpr, pl.ds((i % cpr) * 16, 16)] = zero
    return 0

  lax.fori_loop(0, rows * cpr, body, 0)


def _synth_tail(buf, off):
  lanes = lax.iota(jnp.int32, 16)
  for i in range(SYNTH // 16):
    buf[pl.ds(off + i * 16, 16)] = lanes + (N + i * 16)


def _newton_rsqrt(d):
  half = d * 0.5
  i = plsc.bitcast(d, jnp.int32)
  i = jnp.int32(0x5F3759DF) - lax.shift_right_logical(i, 1)
  y = plsc.bitcast(i, jnp.float32)
  for _ in range(3):
    y = y * (1.5 - half * y * y)
  return y


def _make_fused_conv1_kernel():
  C = 16
  PACK = NPAD // 16
  PROWS = PACK // NS

  @functools.partial(
      pl.kernel,
      mesh=_mesh(),
      out_type=(jax.ShapeDtypeStruct((NC, NPAD, C), jnp.float32),
                jax.ShapeDtypeStruct((NPAD, C), jnp.float32),
                jax.ShapeDtypeStruct((NPAD, C), jnp.float32)),
      compiler_params=pltpu.CompilerParams(use_tc_tiling_on_sc=False,
                                           needs_layout_passes=False),
      scratch_types=[
          pltpu.VMEM((2, PER_TILE), jnp.int32),
          pltpu.VMEM((PER_TILE,), jnp.int32),
          pltpu.VMEM((PACK, 16), jnp.float32),
          pltpu.VMEM((ROWS_PER_TILE, C), jnp.float32),
          pltpu.VMEM((ROWS_PER_TILE, C), jnp.float32),
          pltpu.VMEM((ROWS_PER_TILE, C), jnp.float32),
          pltpu.VMEM((4, CHUNK, C), jnp.float32),
          pltpu.VMEM((CHUNK, C), jnp.float32),
          pltpu.VMEM((PACK // CHUNK, CHUNK), jnp.int32),
          pltpu.VMEM_SHARED((PACK, 16), jnp.float32),
          pltpu.VMEM_SHARED((NPAD, C), jnp.float32),
          pltpu.VMEM_SHARED((NPAD, C), jnp.float32),
          [pltpu.SemaphoreType.DMA] * 4,
          [pltpu.SemaphoreType.DMA] * 4,
      ],
  )
  def fused_kernel(xw_hbm, edge_hbm,
                   acc_hbm, dinv_hbm, y1_hbm,
                   dst_v, src_v, deg_v, xw_v, y1_v, dinv_v, rows_v, zbuf,
                   iid_v, degp_sh, y1_sh, acc_sh, gsems, ssems):
    cid = lax.axis_index("c")
    sid = lax.axis_index("s")
    wid = cid * NS + sid
    nbase = sid * ROWS_PER_TILE

    for h in range(2):
      pltpu.sync_copy(edge_hbm.at[1, pl.ds((h * NS + sid) * EPT, EPT)],
                      dst_v.at[h, pl.ds(0, EPT)])
    pltpu.sync_copy(edge_hbm.at[0, pl.ds(wid * EPT, EPT)],
                    src_v.at[pl.ds(0, EPT)])
    pltpu.sync_copy(xw_hbm.at[pl.ds(nbase, ROWS_PER_TILE)], xw_v)

    lanes = lax.iota(jnp.int32, 16)
    for h in range(2):
      for i in range(SYNTH // 16):
        dst_v[h, pl.ds(EPT + i * 16, 16)] = lanes + (N + i * 16)
    _synth_tail(src_v, EPT)

    _zero_fill(zbuf, CHUNK, C)
    _zero_fill(deg_v, PACK, 16)

    def fill_iid(i, _):
      iid_v[i // 8, pl.ds((i % 8) * 16, 16)] = lanes + i * 16
      return 0

    lax.fori_loop(0, PACK // 16, fill_iid, 0)

    pltpu.sync_copy(zbuf.at[pl.ds(0, PROWS)], degp_sh.at[pl.ds(sid * PROWS,
                                                               PROWS)])

    def zseg(j, _):
      pltpu.sync_copy(zbuf, acc_sh.at[pl.ds(nbase + j * CHUNK, CHUNK)])
      return 0

    lax.fori_loop(0, ROWS_PER_TILE // CHUNK, zseg, 0)

    ones16 = jnp.full((16,), 1.0, jnp.float32)
    for h in range(2):

      def dcount(i, _, h=h):
        ids = dst_v[h, pl.ds(i * 16, 16)]
        plsc.addupdate_scatter(
            deg_v, [lax.shift_right_logical(ids, 4), ids & 15], ones16)
        return 0

      lax.fori_loop(0, PER_TILE // 16, dcount, 0)

    plsc.subcore_barrier()

    def comb(r, _):
      pltpu.sync_copy(deg_v.at[pl.ds(r * CHUNK, CHUNK)],
                      degp_sh.at[iid_v.at[r]], add=True)
      return 0

    lax.fori_loop(0, PACK // CHUNK, comb, 0)
    plsc.subcore_barrier()

    pltpu.sync_copy(degp_sh.at[pl.ds(sid * PROWS, PROWS)],
                    deg_v.at[pl.ds(0, PROWS)])

    def newt(r, _):
      deg_v[r, pl.ds(0, 16)] = _newton_rsqrt(deg_v[r, pl.ds(0, 16)] + 1.0)
      return 0

    lax.fori_loop(0, PROWS, newt, 0)

    def expand(i, _):
      ridx = jnp.full((16,), 0, jnp.int32) + lax.shift_right_logical(i, 4)
      cidx = jnp.full((16,), 0, jnp.int32) + (i & 15)
      dvec = plsc.load_gather(deg_v, [ridx, cidx])
      dinv_v[i, pl.ds(0, 16)] = dvec
      y1_v[i, pl.ds(0, 16)] = xw_v[i, pl.ds(0, 16)] * dvec
      return 0

    lax.fori_loop(0, ROWS_PER_TILE, expand, 0)

    pltpu.sync_copy(y1_v, y1_sh.at[pl.ds(nbase, ROWS_PER_TILE)])

    @pl.when(cid == 0)
    def _():
      pltpu.sync_copy(y1_v, y1_hbm.at[pl.ds(nbase, ROWS_PER_TILE)])
      pltpu.sync_copy(dinv_v, dinv_hbm.at[pl.ds(nbase, ROWS_PER_TILE)])

    plsc.subcore_barrier()

    def issue_g(j, b):
      pltpu.async_copy(y1_sh.at[src_v.at[pl.ds(j * CHUNK, CHUNK)]],
                       rows_v.at[b], gsems[b])

    def wait_g(j, b):
      pltpu.make_async_copy(y1_sh.at[src_v.at[pl.ds(j * CHUNK, CHUNK)]],
                            rows_v.at[b], gsems[b]).wait()

    def issue_s(j, b):
      pltpu.async_copy(rows_v.at[b],
                       acc_sh.at[dst_v.at[cid, pl.ds(j * CHUNK, CHUNK)]],
                       ssems[b], add=True)

    def wait_s(j, b):
      pltpu.make_async_copy(rows_v.at[b],
                            acc_sh.at[dst_v.at[cid, pl.ds(j * CHUNK, CHUNK)]],
                            ssems[b]).wait()

    issue_g(0, 0)
    issue_g(1, 1)

    def group(g, _):
      for b in range(4):
        t = g * 4 + b
        bw = (b + 2) % 4

        @pl.when(t >= 2)
        def _():
          wait_s(t - 2, bw)

        @pl.when(t + 2 < NCHUNK)
        def _():
          issue_g(t + 2, bw)

        wait_g(t, b)
        issue_s(t, b)
      return 0

    lax.fori_loop(0, NCHUNK // 4, group, 0)
    wait_s(NCHUNK - 2, (NCHUNK - 2) % 4)
    wait_s(NCHUNK - 1, (NCHUNK - 1) % 4)
    plsc.subcore_barrier()

    pltpu.sync_copy(
        acc_sh.at[pl.ds(nbase, ROWS_PER_TILE)],
        acc_hbm.at[cid, pl.ds(nbase, ROWS_PER_TILE)])

  return fused_kernel


def _make_conv_kernel(C):
  @functools.partial(
      pl.kernel,
      mesh=_mesh(),
      out_type=jax.ShapeDtypeStruct((NC, NPAD, C), jnp.float32),
      compiler_params=pltpu.CompilerParams(use_tc_tiling_on_sc=False),
      scratch_types=[
          pltpu.VMEM((PER_TILE,), jnp.int32),
          pltpu.VMEM((PER_TILE,), jnp.int32),
          pltpu.VMEM((4, CHUNK, C), jnp.float32),
          pltpu.VMEM((CHUNK, C), jnp.float32),
          pltpu.VMEM_SHARED((NPAD, C), jnp.float32),
          [pltpu.SemaphoreType.DMA] * 4,
          [pltpu.SemaphoreType.DMA] * 4,
      ],
  )
  def conv_kernel(y_hbm, edge_hbm, out_hbm,
                  src_v, dst_v, rows_v, zbuf, acc_sh, gsems, ssems):
    cid = lax.axis_index("c")
    sid = lax.axis_index("s")
    wid = cid * NS + sid
    nbase = sid * ROWS_PER_TILE

    pltpu.sync_copy(edge_hbm.at[0, pl.ds(wid * EPT, EPT)],
                    src_v.at[pl.ds(0, EPT)])
    pltpu.sync_copy(edge_hbm.at[1, pl.ds(wid * EPT, EPT)],
                    dst_v.at[pl.ds(0, EPT)])
    _synth_tail(src_v, EPT)
    _synth_tail(dst_v, EPT)

    _zero_fill(zbuf, CHUNK, C)

    def zseg(j, _):
      pltpu.sync_copy(zbuf, acc_sh.at[pl.ds(nbase + j * CHUNK, CHUNK)])
      return 0

    lax.fori_loop(0, ROWS_PER_TILE // CHUNK, zseg, 0)
    plsc.subcore_barrier()

    def issue_g(j, b):
      pltpu.async_copy(y_hbm.at[src_v.at[pl.ds(j * CHUNK, CHUNK)]],
                       rows_v.at[b], gsems[b])

    def wait_g(j, b):
      pltpu.make_async_copy(y_hbm.at[src_v.at[pl.ds(j * CHUNK, CHUNK)]],
                            rows_v.at[b], gsems[b]).wait()

    def issue_s(j, b):
      pltpu.async_copy(rows_v.at[b],
                       acc_sh.at[dst_v.at[pl.ds(j * CHUNK, CHUNK)]],
                       ssems[b], add=True)

    def wait_s(j, b):
      pltpu.make_async_copy(rows_v.at[b],
                            acc_sh.at[dst_v.at[pl.ds(j * CHUNK, CHUNK)]],
                            ssems[b]).wait()

    issue_g(0, 0)
    issue_g(1, 1)

    def group(g, _):
      for b in range(4):
        t = g * 4 + b
        bw = (b + 2) % 4

        @pl.when(t >= 2)
        def _():
          wait_s(t - 2, bw)

        @pl.when(t + 2 < NCHUNK)
        def _():
          issue_g(t + 2, bw)

        wait_g(t, b)
        issue_s(t, b)
      return 0

    lax.fori_loop(0, NCHUNK // 4, group, 0)
    wait_s(NCHUNK - 2, (NCHUNK - 2) % 4)
    wait_s(NCHUNK - 1, (NCHUNK - 1) % 4)
    plsc.subcore_barrier()

    pltpu.sync_copy(
        acc_sh.at[pl.ds(nbase, ROWS_PER_TILE)],
        out_hbm.at[cid, pl.ds(nbase, ROWS_PER_TILE)])

  return conv_kernel


_fused1 = _make_fused_conv1_kernel()
_conv32 = _make_conv_kernel(32)


def _tc0_body(x_ref, w1_ref, xw_ref):
  xw_ref[0:N, :] = jnp.dot(x_ref[...], w1_ref[...],
                           preferred_element_type=jnp.float32)
  xw_ref[N:NPAD, :] = jnp.zeros((NPAD - N, 16), jnp.float32)


def _tc2_body(accp_ref, y1_ref, dinv_ref, w2_ref, cvec_ref, y2_ref):
  acc = accp_ref[0] + accp_ref[1] + y1_ref[...]
  conv = acc * dinv_ref[...] + cvec_ref[0:1, :]
  h = jnp.maximum(conv * cvec_ref[1:2, :] + cvec_ref[2:3, :], 0.0)
  h2 = jnp.dot(h, w2_ref[...], preferred_element_type=jnp.float32)
  y2_ref[...] = h2 * dinv_ref[:, 0:1]


def _tc3_body(accp_ref, y2_ref, dinv_ref, batch_ref, cvec_ref,
              linw_ref, linb_ref, out_ref):
  acc = accp_ref[0] + accp_ref[1] + y2_ref[...]
  conv = acc * dinv_ref[:, 0:1] + cvec_ref[0:1, :]
  h = jnp.maximum(conv * cvec_ref[1:2, :] + cvec_ref[2:3, :], 0.0)
  ones_col = jnp.ones((NPAD, 1), jnp.float32)
  he = jnp.concatenate([h, ones_col], axis=1)
  gids = lax.broadcasted_iota(jnp.int32, (64, NPAD), 0)
  p = (batch_ref[...] == gids).astype(jnp.float32)
  se = jnp.dot(p, he, preferred_element_type=jnp.float32)
  pooled = se[:, :32] / jnp.maximum(se[:, 32:33], 1.0)
  out_ref[...] = jnp.dot(pooled, linw_ref[...],
                         preferred_element_type=jnp.float32) + linb_ref[...]


def kernel(x, edge_index, batch, W1, b1, bn1_w, bn1_b, W2, b2, bn2_w, bn2_b,
           lin_W, lin_b):
  f32 = jnp.float32
  edge32 = edge_index.astype(jnp.int32)
  batch_p = jnp.concatenate(
      [batch.astype(jnp.int32), jnp.full((NPAD - N,), 64, jnp.int32)])
  batch_p = batch_p.reshape(1, NPAD)

  bn_scale1 = bn1_w * (1.0 / jnp.sqrt(1.0 + EPS))
  bn_scale2 = bn2_w * (1.0 / jnp.sqrt(1.0 + EPS))
  cvec1 = jnp.stack([b1, bn_scale1, bn1_b])
  cvec2 = jnp.stack([b2, bn_scale2, bn2_b])

  xw1 = pl.pallas_call(
      _tc0_body,
      out_shape=jax.ShapeDtypeStruct((NPAD, 16), f32),
  )(x, W1)

  acc1, dinv, y1 = _fused1(xw1, edge32)

  y2 = pl.pallas_call(
      _tc2_body,
      out_shape=jax.ShapeDtypeStruct((NPAD, 32), f32),
  )(acc1, y1, dinv, W2, cvec1)

  acc2 = _conv32(y2, edge32)

  out = pl.pallas_call(
      _tc3_body,
      out_shape=jax.ShapeDtypeStruct((64, 64), f32),
  )(acc2, y2, dinv, batch_p, cvec2, lin_W, lin_b.reshape(1, 64))

  return out
